# Initial kernel scaffold; baseline (speedup 1.0000x reference)
import jax
import jax.numpy as jnp
from jax.experimental import pallas as pl


def kernel(token_ids, weight):
    raise NotImplementedError("write your pallas kernel here")


# speedup vs baseline: 1.4351x; 1.4351x over previous
import functools

import jax
import jax.numpy as jnp
from jax import lax
from jax.experimental import pallas as pl
from jax.experimental.pallas import tpu as pltpu
from jax.experimental.pallas import tpu_sc as plsc

NC = 2
NS = 16
NW = NC * NS
CHUNK = 128


@functools.partial(jax.jit, static_argnums=(2, 3))
def _gather_sc(idx3, weight, n_per_w, n_chunks):
    D = weight.shape[1]
    N = NW * n_per_w
    mesh = plsc.VectorSubcoreMesh(core_axis_name="c", subcore_axis_name="s")

    @functools.partial(
        pl.kernel,
        mesh=mesh,
        compiler_params=pltpu.CompilerParams(use_tc_tiling_on_sc=False),
        out_type=jax.ShapeDtypeStruct((N, D), jnp.float32),
        scratch_types=[
            pltpu.VMEM((n_chunks, CHUNK), jnp.int32),
            pltpu.VMEM((CHUNK, D), jnp.float32),
            pltpu.SemaphoreType.DMA,
        ],
    )
    def k(idx_hbm, table_hbm, out_hbm, idx_v, rows_v, gsem):
        wid = lax.axis_index("s") * NC + lax.axis_index("c")
        base = wid * n_per_w
        pltpu.sync_copy(idx_hbm.at[wid], idx_v)

        def chunk(j, carry):
            pltpu.async_copy(table_hbm.at[idx_v.at[j]], rows_v, gsem).wait()
            pltpu.sync_copy(rows_v, out_hbm.at[pl.ds(base + j * CHUNK, CHUNK)])
            return carry

        lax.fori_loop(0, n_chunks, chunk, 0)

    return k(idx3, weight)


def kernel(token_ids, weight):
    B, F = token_ids.shape
    N = B * F
    assert N % (NW * CHUNK) == 0
    n_per_w = N // NW
    n_chunks = n_per_w // CHUNK
    idx3 = token_ids.astype(jnp.int32).reshape(NW, n_chunks, CHUNK)
    out = _gather_sc(idx3, weight, n_per_w, n_chunks)
    return out.reshape(B, F, weight.shape[1])


# speedup vs baseline: 1.5395x; 1.0727x over previous
import functools

import jax
import jax.numpy as jnp
from jax import lax
from jax.experimental import pallas as pl
from jax.experimental.pallas import tpu as pltpu
from jax.experimental.pallas import tpu_sc as plsc

NC = 2
NS = 16
NW = NC * NS
CHUNK = 512


@functools.partial(jax.jit, static_argnums=(2, 3))
def _gather_sc(idx3, weight, n_per_w, n_chunks):
    D = weight.shape[1]
    N = NW * n_per_w
    mesh = plsc.VectorSubcoreMesh(core_axis_name="c", subcore_axis_name="s")

    @functools.partial(
        pl.kernel,
        mesh=mesh,
        compiler_params=pltpu.CompilerParams(use_tc_tiling_on_sc=False),
        out_type=jax.ShapeDtypeStruct((N, D), jnp.float32),
        scratch_types=[
            pltpu.VMEM((n_chunks, CHUNK), jnp.int32),
            pltpu.VMEM((CHUNK, D), jnp.float32),
            pltpu.SemaphoreType.DMA,
        ],
    )
    def k(idx_hbm, table_hbm, out_hbm, idx_v, rows_v, gsem):
        wid = lax.axis_index("s") * NC + lax.axis_index("c")
        base = wid * n_per_w
        pltpu.sync_copy(idx_hbm.at[wid], idx_v)

        def chunk(j, carry):
            pltpu.async_copy(table_hbm.at[idx_v.at[j]], rows_v, gsem).wait()
            pltpu.sync_copy(rows_v, out_hbm.at[pl.ds(base + j * CHUNK, CHUNK)])
            return carry

        lax.fori_loop(0, n_chunks, chunk, 0)

    return k(idx3, weight)


def kernel(token_ids, weight):
    B, F = token_ids.shape
    N = B * F
    assert N % (NW * CHUNK) == 0
    n_per_w = N // NW
    n_chunks = n_per_w // CHUNK
    idx3 = token_ids.astype(jnp.int32).reshape(NW, n_chunks, CHUNK)
    out = _gather_sc(idx3, weight, n_per_w, n_chunks)
    return out.reshape(B, F, weight.shape[1])


# speedup vs baseline: 1.5684x; 1.0188x over previous
import functools

import jax
import jax.numpy as jnp
from jax import lax
from jax.experimental import pallas as pl
from jax.experimental.pallas import tpu as pltpu
from jax.experimental.pallas import tpu_sc as plsc

NC = 2
NS = 16
NW = NC * NS
CHUNK = 1664


@functools.partial(jax.jit, static_argnums=(2, 3))
def _gather_sc(idx3, weight, n_per_w, n_chunks):
    D = weight.shape[1]
    N = NW * n_per_w
    mesh = plsc.VectorSubcoreMesh(core_axis_name="c", subcore_axis_name="s")

    @functools.partial(
        pl.kernel,
        mesh=mesh,
        compiler_params=pltpu.CompilerParams(use_tc_tiling_on_sc=False),
        out_type=jax.ShapeDtypeStruct((N, D), jnp.float32),
        scratch_types=[
            pltpu.VMEM((n_chunks, CHUNK), jnp.int32),
            pltpu.VMEM((CHUNK, D), jnp.float32),
            pltpu.SemaphoreType.DMA,
        ],
    )
    def k(idx_hbm, table_hbm, out_hbm, idx_v, rows_v, gsem):
        wid = lax.axis_index("s") * NC + lax.axis_index("c")
        base = wid * n_per_w
        pltpu.sync_copy(idx_hbm.at[wid], idx_v)

        def chunk(j, carry):
            pltpu.async_copy(table_hbm.at[idx_v.at[j]], rows_v, gsem).wait()
            pltpu.sync_copy(rows_v, out_hbm.at[pl.ds(base + j * CHUNK, CHUNK)])
            return carry

        lax.fori_loop(0, n_chunks, chunk, 0)

    return k(idx3, weight)


def kernel(token_ids, weight):
    B, F = token_ids.shape
    N = B * F
    assert N % (NW * CHUNK) == 0
    n_per_w = N // NW
    n_chunks = n_per_w // CHUNK
    idx3 = token_ids.astype(jnp.int32).reshape(NW, n_chunks, CHUNK)
    out = _gather_sc(idx3, weight, n_per_w, n_chunks)
    return out.reshape(B, F, weight.shape[1])


# speedup vs baseline: 1.5778x; 1.0060x over previous
import functools

import jax
import jax.numpy as jnp
from jax import lax
from jax.experimental import pallas as pl
from jax.experimental.pallas import tpu as pltpu
from jax.experimental.pallas import tpu_sc as plsc

NC = 2
NS = 16
NW = NC * NS
CHUNK = 1664


@functools.partial(jax.jit, static_argnums=(2, 3))
def _gather_sc(idx3, weight, n_per_w, n_chunks):
    D = weight.shape[1]
    N = NW * n_per_w
    mesh = plsc.VectorSubcoreMesh(core_axis_name="c", subcore_axis_name="s")

    @functools.partial(
        pl.kernel,
        mesh=mesh,
        compiler_params=pltpu.CompilerParams(use_tc_tiling_on_sc=False),
        out_type=jax.ShapeDtypeStruct((N, D), jnp.float32),
        scratch_types=[
            pltpu.VMEM((n_chunks, CHUNK), jnp.int32),
            pltpu.VMEM((2, CHUNK, D), jnp.float32),
            pltpu.SemaphoreType.DMA,
            pltpu.SemaphoreType.DMA,
        ],
    )
    def k(idx_hbm, table_hbm, out_hbm, idx_v, rows_v, gsem, wsem):
        wid = lax.axis_index("s") * NC + lax.axis_index("c")
        base = wid * n_per_w
        pltpu.sync_copy(idx_hbm.at[wid], idx_v)

        gcp = [None] * n_chunks
        wcp = [None] * n_chunks
        gcp[0] = pltpu.async_copy(table_hbm.at[idx_v.at[0]], rows_v.at[0], gsem)
        for j in range(n_chunks):
            nb = j + 1
            if nb < n_chunks:
                if nb >= 2:
                    wcp[nb - 2].wait()
                gcp[nb] = pltpu.async_copy(
                    table_hbm.at[idx_v.at[nb]], rows_v.at[nb % 2], gsem
                )
            gcp[j].wait()
            wcp[j] = pltpu.async_copy(
                rows_v.at[j % 2], out_hbm.at[pl.ds(base + j * CHUNK, CHUNK)], wsem
            )
        wcp[n_chunks - 2].wait()
        wcp[n_chunks - 1].wait()

    return k(idx3, weight)


def kernel(token_ids, weight):
    B, F = token_ids.shape
    N = B * F
    assert N % (NW * CHUNK) == 0
    n_per_w = N // NW
    n_chunks = n_per_w // CHUNK
    idx3 = token_ids.astype(jnp.int32).reshape(NW, n_chunks, CHUNK)
    out = _gather_sc(idx3, weight, n_per_w, n_chunks)
    return out.reshape(B, F, weight.shape[1])
